# baseline (speedup 1.0000x reference)
import functools

import jax
import jax.numpy as jnp
from jax import lax
from jax.experimental import pallas as pl
from jax.experimental.pallas import tpu as pltpu
from jax.experimental.pallas import tpu_sc as plsc

VOCAB = 1000
BATCH = 4096
SEQ = 20
B = BATCH * SEQ
NW = 32
BPW = B // NW
K = 16
CH = BPW // K
NBUF = 4

_mesh = plsc.VectorSubcoreMesh(core_axis_name="c", subcore_axis_name="s")


@functools.partial(
    pl.kernel,
    mesh=_mesh,
    compiler_params=pltpu.CompilerParams(use_tc_tiling_on_sc=False),
    out_type=jax.ShapeDtypeStruct((B, VOCAB), jnp.float32),
    scratch_types=[
        pltpu.VMEM((BPW,), jnp.int32),
        pltpu.VMEM((NBUF, K, VOCAB), jnp.float32),
        pltpu.VMEM_SHARED((VOCAB, VOCAB), jnp.float32),
        [pltpu.SemaphoreType.DMA] * NBUF,
        [pltpu.SemaphoreType.DMA] * NBUF,
    ],
)
def _gather_kernel(table_hbm, idx_hbm, out_hbm, idx_v, bufs, table_sp,
                   gsems, ssems):
    cid = lax.axis_index("c")
    sid = lax.axis_index("s")
    wid = sid * 2 + cid
    base = wid * BPW
    pltpu.sync_copy(idx_hbm.at[pl.ds(base, BPW)], idx_v)

    @pl.when(sid < 10)
    def _load_table():
        pltpu.sync_copy(
            table_hbm.at[pl.ds(sid * 100, 100)],
            table_sp.at[pl.ds(sid * 100, 100)],
        )

    plsc.subcore_barrier()

    def gstart(b, j):
        pltpu.async_copy(
            table_sp.at[idx_v.at[pl.ds(j * K, K)]], bufs.at[b], gsems[b]
        )

    def gwait(b):
        pltpu.make_async_copy(
            table_sp.at[idx_v.at[pl.ds(0, K)]], bufs.at[b], gsems[b]
        ).wait()

    def sstart(b, j):
        pltpu.async_copy(
            bufs.at[b], out_hbm.at[pl.ds(base + j * K, K)], ssems[b]
        )

    def swait(b):
        pltpu.make_async_copy(
            bufs.at[b], out_hbm.at[pl.ds(base, K)], ssems[b]
        ).wait()

    gstart(0, 0)
    for j in range(NBUF - 1):
        b = j % NBUF
        gwait(b)
        sstart(b, j)
        gstart((j + 1) % NBUF, j + 1)

    def group(q, carry):
        j0 = (NBUF - 1) + q * NBUF
        for r in range(NBUF):
            b = (NBUF - 1 + r) % NBUF
            gwait(b)
            sstart(b, j0 + r)
            swait((b + 1) % NBUF)
            gstart((b + 1) % NBUF, j0 + r + 1)
        return carry

    lax.fori_loop(0, (CH - NBUF) // NBUF, group, 0)

    b_last = (CH - 1) % NBUF
    gwait(b_last)
    sstart(b_last, CH - 1)
    for b in range(NBUF):
        swait(b)


def kernel(x_ids, logits_table):
    idx = x_ids.reshape(-1).astype(jnp.int32)
    out = _gather_kernel(logits_table, idx)
    return out.reshape(x_ids.shape + (VOCAB,))
